# baseline (speedup 1.0000x reference)
import functools

import jax
import jax.numpy as jnp
from jax.experimental import pallas as pl
from jax.experimental.pallas import tpu as pltpu


def _pick_vt(v: int) -> int:
    best = None
    for d in range(128, 769, 128):
        if v % d == 0 and (v // d) % 2 == 0:
            best = d
    assert best is not None, v
    return best


def _stats_kernel(x_ref, wg_ref, bg_ref, m_out, s_out, m_sc, s_sc, *,
                  nl: int, l: int):
    k = pl.program_id(1)
    nk = pl.num_programs(1)

    @pl.when(k == 0)
    def _():
        m_sc[...] = jnp.full_like(m_sc, -1e30)
        s_sc[...] = jnp.zeros_like(s_sc)

    wb = wg_ref[...].astype(jnp.bfloat16)
    bg = bg_ref[...]
    for r in range(nl):
        sl = pl.ds(r * l, l)
        logits = jnp.dot(x_ref[sl, :].astype(jnp.bfloat16), wb,
                         preferred_element_type=jnp.float32) + bg
        m_old = m_sc[sl, :]
        m_new = jnp.maximum(m_old, jnp.max(logits, axis=-1, keepdims=True))
        s_sc[sl, :] = (s_sc[sl, :] * jnp.exp(m_old - m_new)
                       + jnp.sum(jnp.exp(logits - m_new), axis=-1, keepdims=True))
        m_sc[sl, :] = m_new

    @pl.when(k == nk - 1)
    def _():
        m_out[0] = m_sc[...]
        s_out[0] = s_sc[...]


def _blend_kernel(x_ref, wg_ref, bg_ref, attn_ref, ids_ref, wc_ref, bc_ref,
                  m2_ref, s2_ref, o_ref, *, nl: int, l: int, s: int, vt: int):
    c = pl.program_id(0)
    k = pl.program_id(1)
    nk = pl.num_programs(1)
    v0 = (c * nk + k) * vt

    wb = wg_ref[...].astype(jnp.bfloat16)
    bg = bg_ref[...]
    for r in range(nl):
        sl = pl.ds(r * l, l)
        xr = x_ref[sl, :]
        m0 = m2_ref[0, sl, :]
        m1 = m2_ref[1, sl, :]
        m = jnp.maximum(m0, m1)
        se = (s2_ref[0, sl, :] * jnp.exp(m0 - m)
              + s2_ref[1, sl, :] * jnp.exp(m1 - m))
        gate = jax.nn.sigmoid(
            jnp.sum(xr * wc_ref[...], axis=-1, keepdims=True) + bc_ref[0, 0])
        q = m - jnp.log(gate / se)
        logits = jnp.dot(xr.astype(jnp.bfloat16), wb,
                         preferred_element_type=jnp.float32) + bg
        probs_scaled = jnp.exp(logits - q)
        iota = jax.lax.broadcasted_iota(jnp.int32, (s, vt), 1) + v0
        onehot = jnp.where(ids_ref[r] == iota, 1.0, 0.0).astype(jnp.bfloat16)
        attn_sc = ((1.0 - gate) * attn_ref[sl, :]).astype(jnp.bfloat16)
        copy_tile = jnp.dot(attn_sc, onehot, preferred_element_type=jnp.float32)
        o_ref[sl, :] = probs_scaled + copy_tile


def kernel(decoder_states, attn_copy, src_token_ids, w_copy, b_copy, w_gen, b_gen):
    n, l, d = decoder_states.shape
    s = attn_copy.shape[-1]
    v = w_gen.shape[-1]
    vt = _pick_vt(v)
    kt = (v // vt) // 2
    rows = n * l

    x = decoder_states.reshape(rows, d)
    attn = attn_copy.reshape(rows, s)
    ids = src_token_ids.astype(jnp.int32).reshape(n, s, 1)
    wc_row = w_copy.reshape(1, d)
    bc = b_copy.reshape(1, 1)
    bg = b_gen.reshape(1, v)

    halves = jax.ShapeDtypeStruct((2, rows, 1), jnp.float32)
    m2, s2 = pl.pallas_call(
        functools.partial(_stats_kernel, nl=n, l=l),
        grid=(2, kt),
        in_specs=[
            pl.BlockSpec((rows, d), lambda c, k: (0, 0)),
            pl.BlockSpec((d, vt), lambda c, k: (0, c * kt + k)),
            pl.BlockSpec((1, vt), lambda c, k: (0, c * kt + k)),
        ],
        out_specs=[
            pl.BlockSpec((1, rows, 1), lambda c, k: (c, 0, 0)),
            pl.BlockSpec((1, rows, 1), lambda c, k: (c, 0, 0)),
        ],
        out_shape=[halves, halves],
        scratch_shapes=[
            pltpu.VMEM((rows, 1), jnp.float32),
            pltpu.VMEM((rows, 1), jnp.float32),
        ],
        compiler_params=pltpu.CompilerParams(
            dimension_semantics=("parallel", "arbitrary"),
            vmem_limit_bytes=50 * 1024 * 1024,
        ),
    )(x, w_gen, bg)

    out = pl.pallas_call(
        functools.partial(_blend_kernel, nl=n, l=l, s=s, vt=vt),
        grid=(2, kt),
        in_specs=[
            pl.BlockSpec((rows, d), lambda c, k: (0, 0)),
            pl.BlockSpec((d, vt), lambda c, k: (0, c * kt + k)),
            pl.BlockSpec((1, vt), lambda c, k: (0, c * kt + k)),
            pl.BlockSpec((rows, s), lambda c, k: (0, 0)),
            pl.BlockSpec((n, s, 1), lambda c, k: (0, 0, 0)),
            pl.BlockSpec((1, d), lambda c, k: (0, 0)),
            pl.BlockSpec((1, 1), lambda c, k: (0, 0)),
            pl.BlockSpec((2, rows, 1), lambda c, k: (0, 0, 0)),
            pl.BlockSpec((2, rows, 1), lambda c, k: (0, 0, 0)),
        ],
        out_specs=pl.BlockSpec((rows, vt), lambda c, k: (0, c * kt + k)),
        out_shape=jax.ShapeDtypeStruct((rows, v), jnp.float32),
        compiler_params=pltpu.CompilerParams(
            dimension_semantics=("parallel", "arbitrary"),
            vmem_limit_bytes=50 * 1024 * 1024,
        ),
    )(x, w_gen, bg, attn, ids, wc_row, bc, m2, s2)

    return out.reshape(n, l, v)
